# baseline (speedup 1.0000x reference)
import functools

import jax
import jax.numpy as jnp
from jax import lax
from jax.experimental import pallas as pl
from jax.experimental.pallas import tpu as pltpu
from jax.experimental.pallas import tpu_sc as plsc

BATCH = 4096
NUM_CORES = 2
NUM_SUBCORES = 16
NUM_WORKERS = NUM_CORES * NUM_SUBCORES
ROWS_PER_WORKER = BATCH // NUM_WORKERS
CHUNK = 2
NUM_CHUNKS = ROWS_PER_WORKER // CHUNK

_mesh = plsc.VectorSubcoreMesh(core_axis_name="c", subcore_axis_name="s")

_DNUMS = lax.GatherDimensionNumbers(
    offset_dims=(), collapsed_slice_dims=(0,), start_index_map=(0,))


def _splat(vec, i):
    idx = (lax.iota(jnp.int32, 16) * 0 + i).reshape(16, 1)
    return lax.gather(vec, idx, dimension_numbers=_DNUMS,
                      slice_sizes=(1,),
                      mode=lax.GatherScatterMode.PROMISE_IN_BOUNDS)


def _full16(v):
    return jnp.zeros((16,), jnp.int32) + v


@functools.partial(
    pl.kernel,
    mesh=_mesh,
    compiler_params=pltpu.CompilerParams(needs_layout_passes=False),
    out_type=jax.ShapeDtypeStruct((BATCH, 64, 256), jnp.float32),
    scratch_types=[
        pltpu.VMEM((2, CHUNK, 16, 128), jnp.float32),
        pltpu.VMEM((2, CHUNK, 64, 256), jnp.float32),
        pltpu.SemaphoreType.DMA,
        pltpu.SemaphoreType.DMA,
        pltpu.SemaphoreType.DMA,
        pltpu.SemaphoreType.DMA,
    ],
)
def _rat_sc(x_hbm, out_hbm, in_v, out_v, sin0, sin1, sout0, sout1):
    wid = lax.axis_index("s") * NUM_CORES + lax.axis_index("c")
    base = wid * ROWS_PER_WORKER

    def start_in(g, p):
        pltpu.async_copy(
            x_hbm.at[pl.ds(base + g * CHUNK, CHUNK)],
            in_v.at[p], sin0 if p == 0 else sin1)

    def wait_in(p):
        pltpu.make_async_copy(
            x_hbm.at[pl.ds(0, CHUNK)], in_v.at[p],
            sin0 if p == 0 else sin1).wait()

    def start_out(g, p):
        pltpu.async_copy(
            out_v.at[p],
            out_hbm.at[pl.ds(base + g * CHUNK, CHUNK)],
            sout0 if p == 0 else sout1)

    def wait_out(p):
        pltpu.make_async_copy(
            out_v.at[p], out_hbm.at[pl.ds(0, CHUNK)],
            sout0 if p == 0 else sout1).wait()

    def compute(p):
        lane = lax.iota(jnp.int32, 16)
        zero = lane * 0

        def f_body(f, _):
            f2 = 2 * f
            for r in range(CHUNK):
                left = plsc.load_gather(
                    in_v, [zero + p, zero + r, lane, zero + f2])
                right = plsc.load_gather(
                    in_v, [zero + p, zero + r, lane, zero + f2 + 1])
                for i in range(16):
                    out_v[p, r, f, pl.ds(i * 16, 16)] = (
                        _splat(left, i) + right)
            return 0
        lax.fori_loop(0, 64, f_body, 0, unroll=2)

    start_in(0, 0)
    start_in(1, 1)

    def do_parity(g, p):
        wait_in(p)

        @pl.when(g >= 2)
        def _():
            wait_out(p)
        compute(p)
        start_out(g, p)

        @pl.when(g + 2 < NUM_CHUNKS)
        def _():
            start_in(g + 2, p)

    def chunk_body(g, _):
        @pl.when(g % 2 == 0)
        def _():
            do_parity(g, 0)

        @pl.when(g % 2 == 1)
        def _():
            do_parity(g, 1)
        return 0

    lax.fori_loop(0, NUM_CHUNKS, chunk_body, 0)
    wait_out(0)
    wait_out(1)


def kernel(x):
    xt = jnp.transpose(x, (0, 2, 1))
    return _rat_sc(xt)


# speedup vs baseline: 1.0000x; 1.0000x over previous
import functools

import jax
import jax.numpy as jnp
from jax import lax
from jax.experimental import pallas as pl
from jax.experimental.pallas import tpu as pltpu
from jax.experimental.pallas import tpu_sc as plsc

BATCH = 4096
NUM_CORES = 2
NUM_SUBCORES = 16
NUM_WORKERS = NUM_CORES * NUM_SUBCORES
ROWS_PER_WORKER = BATCH // NUM_WORKERS
CHUNK = 2
NUM_CHUNKS = ROWS_PER_WORKER // CHUNK

_mesh = plsc.VectorSubcoreMesh(core_axis_name="c", subcore_axis_name="s")

_DNUMS = lax.GatherDimensionNumbers(
    offset_dims=(), collapsed_slice_dims=(0,), start_index_map=(0,))


def _splat(vec, i):
    idx = (lax.iota(jnp.int32, 16) * 0 + i).reshape(16, 1)
    return lax.gather(vec, idx, dimension_numbers=_DNUMS,
                      slice_sizes=(1,),
                      mode=lax.GatherScatterMode.PROMISE_IN_BOUNDS)


def _full16(v):
    return jnp.zeros((16,), jnp.int32) + v


@functools.partial(
    pl.kernel,
    mesh=_mesh,
    compiler_params=pltpu.CompilerParams(needs_layout_passes=False),
    out_type=jax.ShapeDtypeStruct((BATCH, 64, 256), jnp.float32),
    scratch_types=[
        pltpu.VMEM((2, CHUNK, 16, 129), jnp.float32),
        pltpu.VMEM((2, CHUNK, 64, 256), jnp.float32),
        pltpu.SemaphoreType.DMA,
        pltpu.SemaphoreType.DMA,
        pltpu.SemaphoreType.DMA,
        pltpu.SemaphoreType.DMA,
    ],
)
def _rat_sc(x_hbm, out_hbm, in_v, out_v, sin0, sin1, sout0, sout1):
    wid = lax.axis_index("s") * NUM_CORES + lax.axis_index("c")
    base = wid * ROWS_PER_WORKER

    def start_in(g, p):
        pltpu.async_copy(
            x_hbm.at[pl.ds(base + g * CHUNK, CHUNK)],
            in_v.at[p, :, :, pl.ds(0, 128)], sin0 if p == 0 else sin1)

    def wait_in(p):
        pltpu.make_async_copy(
            x_hbm.at[pl.ds(0, CHUNK)], in_v.at[p, :, :, pl.ds(0, 128)],
            sin0 if p == 0 else sin1).wait()

    def start_out(g, p):
        pltpu.async_copy(
            out_v.at[p],
            out_hbm.at[pl.ds(base + g * CHUNK, CHUNK)],
            sout0 if p == 0 else sout1)

    def wait_out(p):
        pltpu.make_async_copy(
            out_v.at[p], out_hbm.at[pl.ds(0, CHUNK)],
            sout0 if p == 0 else sout1).wait()

    def compute(p):
        lane = lax.iota(jnp.int32, 16)
        zero = lane * 0

        def f_body(f, _):
            f2 = 2 * f
            for r in range(CHUNK):
                left = plsc.load_gather(
                    in_v, [zero + p, zero + r, lane, zero + f2])
                right = plsc.load_gather(
                    in_v, [zero + p, zero + r, lane, zero + f2 + 1])
                for i in range(16):
                    out_v[p, r, f, pl.ds(i * 16, 16)] = (
                        _splat(left, i) + right)
            return 0
        lax.fori_loop(0, 64, f_body, 0, unroll=2)

    start_in(0, 0)
    start_in(1, 1)

    def do_parity(g, p):
        wait_in(p)

        @pl.when(g >= 2)
        def _():
            wait_out(p)
        compute(p)
        start_out(g, p)

        @pl.when(g + 2 < NUM_CHUNKS)
        def _():
            start_in(g + 2, p)

    def chunk_body(g, _):
        @pl.when(g % 2 == 0)
        def _():
            do_parity(g, 0)

        @pl.when(g % 2 == 1)
        def _():
            do_parity(g, 1)
        return 0

    lax.fori_loop(0, NUM_CHUNKS, chunk_body, 0)
    wait_out(0)
    wait_out(1)


def kernel(x):
    xt = jnp.transpose(x, (0, 2, 1))
    return _rat_sc(xt)


# speedup vs baseline: 1.2399x; 1.2399x over previous
import functools

import jax
import jax.numpy as jnp
from jax import lax
from jax.experimental import pallas as pl
from jax.experimental.pallas import tpu as pltpu
from jax.experimental.pallas import tpu_sc as plsc

BATCH = 4096
NUM_CORES = 2
NUM_SUBCORES = 16
NUM_WORKERS = NUM_CORES * NUM_SUBCORES
ROWS_PER_WORKER = BATCH // NUM_WORKERS
CHUNK = 2
NUM_CHUNKS = ROWS_PER_WORKER // CHUNK

_mesh = plsc.VectorSubcoreMesh(core_axis_name="c", subcore_axis_name="s")

_DNUMS = lax.GatherDimensionNumbers(
    offset_dims=(), collapsed_slice_dims=(0,), start_index_map=(0,))


def _splat(vec, i):
    idx = (lax.iota(jnp.int32, 16) * 0 + i).reshape(16, 1)
    return lax.gather(vec, idx, dimension_numbers=_DNUMS,
                      slice_sizes=(1,),
                      mode=lax.GatherScatterMode.PROMISE_IN_BOUNDS)


def _full16(v):
    return jnp.zeros((16,), jnp.int32) + v


@functools.partial(
    pl.kernel,
    mesh=_mesh,
    compiler_params=pltpu.CompilerParams(needs_layout_passes=False),
    out_type=jax.ShapeDtypeStruct((BATCH, 64, 256), jnp.float32),
    scratch_types=[
        pltpu.VMEM((2, CHUNK, 16, 129), jnp.float32),
        pltpu.VMEM((2, CHUNK, 64, 256), jnp.float32),
        pltpu.SemaphoreType.DMA,
        pltpu.SemaphoreType.DMA,
        pltpu.SemaphoreType.DMA,
        pltpu.SemaphoreType.DMA,
    ],
)
def _rat_sc(x_hbm, out_hbm, in_v, out_v, sin0, sin1, sout0, sout1):
    wid = lax.axis_index("s") * NUM_CORES + lax.axis_index("c")
    base = wid * ROWS_PER_WORKER

    def start_in(g, p):
        pltpu.async_copy(
            x_hbm.at[pl.ds(base + g * CHUNK, CHUNK)],
            in_v.at[p, :, :, pl.ds(0, 128)], sin0 if p == 0 else sin1)

    def wait_in(p):
        pltpu.make_async_copy(
            x_hbm.at[pl.ds(0, CHUNK)], in_v.at[p, :, :, pl.ds(0, 128)],
            sin0 if p == 0 else sin1).wait()

    def start_out(g, p):
        pltpu.async_copy(
            out_v.at[p],
            out_hbm.at[pl.ds(base + g * CHUNK, CHUNK)],
            sout0 if p == 0 else sout1)

    def wait_out(p):
        pltpu.make_async_copy(
            out_v.at[p], out_hbm.at[pl.ds(0, CHUNK)],
            sout0 if p == 0 else sout1).wait()

    def compute(p):
        lane = lax.iota(jnp.int32, 16)
        zero = lane * 0

        def g_body(fg, _):
            col = fg * 16
            for r in range(CHUNK):
                nv = [in_v[p, r, i, pl.ds(col, 16)] for i in range(16)]
                for fo in range(8):
                    f = fg * 8 + fo
                    right = plsc.load_gather(
                        in_v,
                        [zero + p, zero + r, lane, zero + col + 2 * fo + 1])
                    for i in range(16):
                        out_v[p, r, f, pl.ds(i * 16, 16)] = (
                            _splat(nv[i], 2 * fo) + right)
            return 0
        lax.fori_loop(0, 8, g_body, 0)

    start_in(0, 0)
    start_in(1, 1)

    def do_parity(g, p):
        wait_in(p)

        @pl.when(g >= 2)
        def _():
            wait_out(p)
        compute(p)
        start_out(g, p)

        @pl.when(g + 2 < NUM_CHUNKS)
        def _():
            start_in(g + 2, p)

    def chunk_body(g, _):
        @pl.when(g % 2 == 0)
        def _():
            do_parity(g, 0)

        @pl.when(g % 2 == 1)
        def _():
            do_parity(g, 1)
        return 0

    lax.fori_loop(0, NUM_CHUNKS, chunk_body, 0)
    wait_out(0)
    wait_out(1)


def kernel(x):
    xt = jnp.transpose(x, (0, 2, 1))
    return _rat_sc(xt)


# speedup vs baseline: 1.9432x; 1.5672x over previous
import functools

import jax
import jax.numpy as jnp
from jax import lax
from jax.experimental import pallas as pl
from jax.experimental.pallas import tpu as pltpu
from jax.experimental.pallas import tpu_sc as plsc

BATCH = 4096
NUM_CORES = 2
NUM_SUBCORES = 16
NUM_WORKERS = NUM_CORES * NUM_SUBCORES
ROWS_PER_WORKER = BATCH // NUM_WORKERS
CHUNK = 2
NUM_CHUNKS = ROWS_PER_WORKER // CHUNK

_mesh = plsc.VectorSubcoreMesh(core_axis_name="c", subcore_axis_name="s")

_DNUMS = lax.GatherDimensionNumbers(
    offset_dims=(), collapsed_slice_dims=(0,), start_index_map=(0,))


def _permute(vec, idx16):
    return lax.gather(vec, idx16.reshape(16, 1), dimension_numbers=_DNUMS,
                      slice_sizes=(1,),
                      mode=lax.GatherScatterMode.PROMISE_IN_BOUNDS)


def _splat(vec, i):
    idx = lax.iota(jnp.int32, 16) * 0 + i
    return _permute(vec, idx)


def _transpose16(vs, lane):
    t = list(vs)
    for s in (1, 2, 4, 8):
        xidx = jnp.bitwise_xor(lane, s)
        keep = (lane & s) == 0
        for r0 in range(16):
            if r0 & s:
                continue
            a, b = t[r0], t[r0 + s]
            pa = _permute(b, xidx)
            pb = _permute(a, xidx)
            t[r0] = jnp.where(keep, a, pa)
            t[r0 + s] = jnp.where(keep, pb, b)
    return t


@functools.partial(
    pl.kernel,
    mesh=_mesh,
    compiler_params=pltpu.CompilerParams(needs_layout_passes=False),
    out_type=jax.ShapeDtypeStruct((BATCH, 64, 256), jnp.float32),
    scratch_types=[
        pltpu.VMEM((2, CHUNK, 16, 129), jnp.float32),
        pltpu.VMEM((2, CHUNK, 64, 256), jnp.float32),
        pltpu.SemaphoreType.DMA,
        pltpu.SemaphoreType.DMA,
        pltpu.SemaphoreType.DMA,
        pltpu.SemaphoreType.DMA,
    ],
)
def _rat_sc(x_hbm, out_hbm, in_v, out_v, sin0, sin1, sout0, sout1):
    wid = lax.axis_index("s") * NUM_CORES + lax.axis_index("c")
    base = wid * ROWS_PER_WORKER

    def start_in(g, p):
        pltpu.async_copy(
            x_hbm.at[pl.ds(base + g * CHUNK, CHUNK)],
            in_v.at[p, :, :, pl.ds(0, 128)], sin0 if p == 0 else sin1)

    def wait_in(p):
        pltpu.make_async_copy(
            x_hbm.at[pl.ds(0, CHUNK)], in_v.at[p, :, :, pl.ds(0, 128)],
            sin0 if p == 0 else sin1).wait()

    def start_out(g, p):
        pltpu.async_copy(
            out_v.at[p],
            out_hbm.at[pl.ds(base + g * CHUNK, CHUNK)],
            sout0 if p == 0 else sout1)

    def wait_out(p):
        pltpu.make_async_copy(
            out_v.at[p], out_hbm.at[pl.ds(0, CHUNK)],
            sout0 if p == 0 else sout1).wait()

    def compute(p):
        lane = lax.iota(jnp.int32, 16)

        def g_body(fg, _):
            col = fg * 16
            for r in range(CHUNK):
                nv = [in_v[p, r, i, pl.ds(col, 16)] for i in range(16)]
                t = _transpose16(nv, lane)
                for fo in range(8):
                    f = fg * 8 + fo
                    right = t[2 * fo + 1]
                    for i in range(16):
                        out_v[p, r, f, pl.ds(i * 16, 16)] = (
                            _splat(nv[i], 2 * fo) + right)
            return 0
        lax.fori_loop(0, 8, g_body, 0)

    start_in(0, 0)
    start_in(1, 1)

    def do_parity(g, p):
        wait_in(p)

        @pl.when(g >= 2)
        def _():
            wait_out(p)
        compute(p)
        start_out(g, p)

        @pl.when(g + 2 < NUM_CHUNKS)
        def _():
            start_in(g + 2, p)

    def chunk_body(g, _):
        @pl.when(g % 2 == 0)
        def _():
            do_parity(g, 0)

        @pl.when(g % 2 == 1)
        def _():
            do_parity(g, 1)
        return 0

    lax.fori_loop(0, NUM_CHUNKS, chunk_body, 0)
    wait_out(0)
    wait_out(1)


def kernel(x):
    xt = jnp.transpose(x, (0, 2, 1))
    return _rat_sc(xt)
